# baseline (speedup 1.0000x reference)
import jax
import jax.numpy as jnp
from jax import lax
from jax.experimental import pallas as pl
from jax.experimental.pallas import tpu as pltpu
from jax.experimental.pallas import tpu_sc as plsc

N = 10000
E = 160000
D = 256
NCHUNK = 4
CW = D // NCHUNK
NB = 1000
EB = 2000
SUBC = 16
EPS_SEG = 1e-6
EPS_BN = 1e-5

B = 80
EPW = E // SUBC
NBLK = EPW // B
ROWS_MAIN = 624
ROWS_TAIL = N - SUBC * ROWS_MAIN
WROWS = 16


def _node_mm_body(x_ref, wa_ref, ba_ref, wb_ref, bb_ref, wd_ref, bd_ref,
                  we_ref, be_ref, ax_ref, ebt_ref, dxt_ref):
    xb = x_ref[...]
    a = jnp.dot(xb, wa_ref[...], preferred_element_type=jnp.float32) + ba_ref[...]
    b = jnp.dot(xb, wb_ref[...], preferred_element_type=jnp.float32) + bb_ref[...]
    d = jnp.dot(xb, wd_ref[...], preferred_element_type=jnp.float32) + bd_ref[...]
    ex = jnp.dot(xb, we_ref[...], preferred_element_type=jnp.float32) + be_ref[...]
    ax_ref[...] = a
    for q in range(NCHUNK):
        ebt_ref[q, :, 0:CW] = ex[:, q * CW:(q + 1) * CW]
        ebt_ref[q, :, CW:2 * CW] = b[:, q * CW:(q + 1) * CW]
    for h in range(2):
        dxt_ref[h, :, :] = d[:, h * 2 * CW:(h + 1) * 2 * CW]


def _node_matmuls(x, WA, bA, WB_, bB, WD, bD, WEw, bEw):
    wspec = pl.BlockSpec((D, D), lambda i: (0, 0))
    bspec = pl.BlockSpec((1, D), lambda i: (0, 0))
    return pl.pallas_call(
        _node_mm_body,
        grid=(N // NB,),
        in_specs=[
            pl.BlockSpec((NB, D), lambda i: (i, 0)),
            wspec, bspec, wspec, bspec, wspec, bspec, wspec, bspec,
        ],
        out_specs=[
            pl.BlockSpec((NB, D), lambda i: (i, 0)),
            pl.BlockSpec((NCHUNK, NB, 2 * CW), lambda i: (0, i, 0)),
            pl.BlockSpec((2, NB, 2 * CW), lambda i: (0, i, 0)),
        ],
        out_shape=[
            jax.ShapeDtypeStruct((N, D), jnp.float32),
            jax.ShapeDtypeStruct((NCHUNK, N, 2 * CW), jnp.float32),
            jax.ShapeDtypeStruct((2, N, 2 * CW), jnp.float32),
        ],
    )(x, WA, bA.reshape(1, D), WB_, bB.reshape(1, D), WD, bD.reshape(1, D),
      WEw, bEw.reshape(1, D))


def _edge_mm_body(e_ref, wc_ref, bc_ref, ce_ref):
    ce = jnp.dot(e_ref[...], wc_ref[...],
                 preferred_element_type=jnp.float32) + bc_ref[...]
    for q in range(NCHUNK):
        ce_ref[q, :, :] = ce[:, q * CW:(q + 1) * CW]


def _edge_matmul(e, WC, bC):
    return pl.pallas_call(
        _edge_mm_body,
        grid=(E // EB,),
        in_specs=[
            pl.BlockSpec((EB, D), lambda i: (i, 0)),
            pl.BlockSpec((D, D), lambda i: (0, 0)),
            pl.BlockSpec((1, D), lambda i: (0, 0)),
        ],
        out_specs=pl.BlockSpec((NCHUNK, EB, CW), lambda i: (0, i, 0)),
        out_shape=jax.ShapeDtypeStruct((NCHUNK, E, CW), jnp.float32),
    )(e, WC, bC.reshape(1, D))


def _sc_edge_kernel(ebt_hbm, dxt_hbm, ce_hbm, src_hbm, dst_hbm,
                    eij_hbm, nd_hbm, stats_hbm,
                    acc, sidx, didx, gsrc, gdst, geb, gd, ceb, eout, vals,
                    wbuf, stloc, sem_eb, sem_d, sem_ce):
    c = lax.axis_index("c")
    s = lax.axis_index("s")

    for p in range(2):
        q = 2 * c + p
        qn = q * N
        qe = q * E

        @pl.loop(0, WROWS)
        def _(r):
            for k in range(2 * CW // 16):
                wbuf[r, pl.ds(k * 16, 16)] = jnp.zeros((16,), jnp.float32)

        @pl.loop(0, ROWS_MAIN // WROWS)
        def _(t):
            pltpu.sync_copy(
                wbuf, acc.at[pl.ds(pl.multiple_of(s * ROWS_MAIN + t * WROWS, 16), WROWS)])

        @pl.when(s == SUBC - 1)
        def _():
            pltpu.sync_copy(wbuf.at[pl.ds(0, ROWS_TAIL)],
                            acc.at[pl.ds(SUBC * ROWS_MAIN, ROWS_TAIL)])

        for k in range(2 * CW // 16):
            stloc[pl.ds(k * 16, 16)] = jnp.zeros((16,), jnp.float32)

        plsc.subcore_barrier()

        @pl.loop(0, NBLK)
        def _(blk):
            base = pl.multiple_of(s * EPW + blk * B, 16)
            pltpu.sync_copy(src_hbm.at[pl.ds(base, B)], sidx)
            pltpu.sync_copy(dst_hbm.at[pl.ds(base, B)], didx)

            @pl.loop(0, B // 16)
            def _(k):
                sl = pl.ds(k * 16, 16)
                gsrc[sl] = sidx[sl] + qn
                gdst[sl] = didx[sl] + c * N

            cp_eb = pltpu.async_copy(ebt_hbm.at[gsrc], geb, sem_eb)
            cp_d = pltpu.async_copy(dxt_hbm.at[gdst], gd, sem_d)
            cp_ce = pltpu.async_copy(
                ce_hbm.at[pl.ds(pl.multiple_of((qe + base) // 2, 8), B // 2)],
                ceb, sem_ce)
            cp_eb.wait()
            cp_d.wait()
            cp_ce.wait()

            @pl.loop(0, B // 2)
            def _(i2):
                for h2 in range(2):
                    i = i2 * 2 + h2
                    for k in range(CW // 16):
                        sl = pl.ds(k * 16, 16)
                        sh = pl.ds(CW + k * 16, 16)
                        sp = pl.ds(h2 * CW + k * 16, 16)
                        z = gd[i, pl.ds(p * CW + k * 16, 16)] + geb[i, sl] + ceb[i2, sp]
                        eout[i2, sp] = z
                        stloc[sl] = stloc[sl] + z
                        stloc[sh] = stloc[sh] + z * z
                        sg = 1.0 / (1.0 + jnp.exp(-z))
                        vals[i, sl] = sg * geb[i, sh]
                        vals[i, sh] = sg

            pltpu.sync_copy(
                eout, eij_hbm.at[pl.ds(pl.multiple_of((qe + base) // 2, 8), B // 2)])
            pltpu.sync_copy(vals, acc.at[didx], add=True)

        plsc.subcore_barrier()

        @pl.loop(0, ROWS_MAIN // WROWS)
        def _(t):
            lo = pl.multiple_of(s * ROWS_MAIN + t * WROWS, 16)
            pltpu.sync_copy(acc.at[pl.ds(lo, WROWS)], wbuf)
            pltpu.sync_copy(wbuf, nd_hbm.at[pl.ds(pl.multiple_of(qn + lo, 16), WROWS)])

        @pl.when(s == SUBC - 1)
        def _():
            pltpu.sync_copy(acc.at[pl.ds(SUBC * ROWS_MAIN, ROWS_TAIL)],
                            wbuf.at[pl.ds(0, ROWS_TAIL)])
            pltpu.sync_copy(wbuf.at[pl.ds(0, ROWS_TAIL)],
                            nd_hbm.at[pl.ds(qn + SUBC * ROWS_MAIN, ROWS_TAIL)])

        pltpu.sync_copy(stloc, stats_hbm.at[q * SUBC + s])

        plsc.subcore_barrier()


def _sc_edge(ebt, dxt, ce4, src, dst):
    mesh = plsc.VectorSubcoreMesh(core_axis_name="c", subcore_axis_name="s")
    kern = pl.kernel(
        _sc_edge_kernel,
        out_type=[
            jax.ShapeDtypeStruct((NCHUNK * E // 2, 2 * CW), jnp.float32),
            jax.ShapeDtypeStruct((NCHUNK * N, 2 * CW), jnp.float32),
            jax.ShapeDtypeStruct((NCHUNK * SUBC, 2 * CW), jnp.float32),
        ],
        mesh=mesh,
        scratch_types=[
            pltpu.VMEM_SHARED((N, 2 * CW), jnp.float32),
            pltpu.VMEM((B,), jnp.int32),
            pltpu.VMEM((B,), jnp.int32),
            pltpu.VMEM((B,), jnp.int32),
            pltpu.VMEM((B,), jnp.int32),
            pltpu.VMEM((B, 2 * CW), jnp.float32),
            pltpu.VMEM((B, 2 * CW), jnp.float32),
            pltpu.VMEM((B // 2, 2 * CW), jnp.float32),
            pltpu.VMEM((B // 2, 2 * CW), jnp.float32),
            pltpu.VMEM((B, 2 * CW), jnp.float32),
            pltpu.VMEM((WROWS, 2 * CW), jnp.float32),
            pltpu.VMEM((2 * CW,), jnp.float32),
            pltpu.SemaphoreType.DMA,
            pltpu.SemaphoreType.DMA,
            pltpu.SemaphoreType.DMA,
        ],
    )
    return kern(ebt.reshape(NCHUNK * N, 2 * CW),
                dxt.reshape(2 * N, 2 * CW),
                ce4.reshape(NCHUNK * E // 2, 2 * CW), src, dst)


def _x_out_body(ax_ref, nd_ref, g_ref, b_ref, o_ref):
    cols = []
    for q in range(NCHUNK):
        num = nd_ref[q, :, 0:CW]
        den = nd_ref[q, :, CW:2 * CW]
        cols.append(ax_ref[:, q * CW:(q + 1) * CW] + num / (den + EPS_SEG))
    xo = jnp.concatenate(cols, axis=1)
    mu = jnp.mean(xo, axis=0, keepdims=True)
    var = jnp.mean(xo * xo, axis=0, keepdims=True) - mu * mu
    y = g_ref[...] * (xo - mu) * lax.rsqrt(var + EPS_BN) + b_ref[...]
    o_ref[...] = jnp.maximum(y, 0.0)


def _x_out(ax, nd, gamma_x, beta_x):
    return pl.pallas_call(
        _x_out_body,
        grid=(1,),
        in_specs=[
            pl.BlockSpec((N, D), lambda i: (0, 0)),
            pl.BlockSpec((NCHUNK, N, 2 * CW), lambda i: (0, 0, 0)),
            pl.BlockSpec((1, D), lambda i: (0, 0)),
            pl.BlockSpec((1, D), lambda i: (0, 0)),
        ],
        out_specs=pl.BlockSpec((N, D), lambda i: (0, 0)),
        out_shape=jax.ShapeDtypeStruct((N, D), jnp.float32),
    )(ax, nd.reshape(NCHUNK, N, 2 * CW), gamma_x.reshape(1, D),
      beta_x.reshape(1, D))


def _e_out_body(eij_ref, st_ref, g_ref, b_ref, o_ref):
    st = st_ref[...].reshape(NCHUNK, SUBC, 2 * CW).sum(axis=1)
    mu = jnp.concatenate([st[q:q + 1, 0:CW] for q in range(NCHUNK)], axis=1) / E
    ex2 = jnp.concatenate(
        [st[q:q + 1, CW:2 * CW] for q in range(NCHUNK)], axis=1) / E
    var = ex2 - mu * mu
    scale = g_ref[...] * lax.rsqrt(var + EPS_BN)
    shift = b_ref[...] - mu * scale
    blk = jnp.concatenate([eij_ref[q] for q in range(NCHUNK)], axis=1)
    o_ref[...] = jnp.maximum(blk * scale + shift, 0.0)


def _e_out(eij4, stats, gamma_e, beta_e):
    return pl.pallas_call(
        _e_out_body,
        grid=(E // EB,),
        in_specs=[
            pl.BlockSpec((NCHUNK, EB, CW), lambda i: (0, i, 0)),
            pl.BlockSpec((NCHUNK * SUBC, 2 * CW), lambda i: (0, 0)),
            pl.BlockSpec((1, D), lambda i: (0, 0)),
            pl.BlockSpec((1, D), lambda i: (0, 0)),
        ],
        out_specs=pl.BlockSpec((EB, D), lambda i: (i, 0)),
        out_shape=jax.ShapeDtypeStruct((E, D), jnp.float32),
    )(eij4.reshape(NCHUNK, E, CW), stats, gamma_e.reshape(1, D),
      beta_e.reshape(1, D))


@jax.jit
def kernel(x, e, edge_index, WA, bA, WB, bB, WC, bC, WD, bD, WEw, bEw,
           gamma_x, beta_x, gamma_e, beta_e):
    ax, ebt, dxt = _node_matmuls(x, WA, bA, WB, bB, WD, bD, WEw, bEw)
    ce4 = _edge_matmul(e, WC, bC)
    src = edge_index[0]
    dst = edge_index[1]
    eij4, nd, stats = _sc_edge(ebt, dxt, ce4, src, dst)
    x_out = _x_out(ax, nd, gamma_x, beta_x)
    e_out = _e_out(eij4, stats, gamma_e, beta_e)
    return (x_out, e_out)
